# Initial kernel scaffold; baseline (speedup 1.0000x reference)
import jax
import jax.numpy as jnp
from jax.experimental import pallas as pl


def kernel(x, embedding_table):
    raise NotImplementedError("write your pallas kernel here")


# speedup vs baseline: 2.4217x; 2.4217x over previous
import functools

import jax
import jax.numpy as jnp
from jax import lax
from jax.experimental import pallas as pl
from jax.experimental.pallas import tpu as pltpu
from jax.experimental.pallas import tpu_sc as plsc

_NC, _NS = 2, 16


def _emb_kernel(n, d, nch, chunk, scale):
    nw = _NC * _NS
    per_w = n // nw

    mesh = plsc.VectorSubcoreMesh(
        core_axis_name="c", subcore_axis_name="s",
        num_cores=_NC, num_subcores=_NS,
    )

    @functools.partial(
        pl.kernel,
        out_type=jax.ShapeDtypeStruct((n, d), jnp.float32),
        mesh=mesh,
        scratch_types=[
            pltpu.VMEM((nch, chunk), jnp.int32),
            pltpu.VMEM((chunk, d), jnp.float32),
            pltpu.SemaphoreType.DMA,
        ],
    )
    def emb(idx_hbm, table_hbm, out_hbm, idx_v, rows_v, sem):
        wid = lax.axis_index("s") * _NC + lax.axis_index("c")
        base = wid * per_w
        pltpu.sync_copy(idx_hbm.at[wid], idx_v)

        def body(c, carry):
            pltpu.async_copy(table_hbm.at[idx_v.at[c]], rows_v, sem).wait()

            def srow(r, carry2):
                for j in range(d // 16):
                    sl = pl.ds(j * 16, 16)
                    rows_v[r, sl] = rows_v[r, sl] * scale
                return carry2

            lax.fori_loop(0, chunk, srow, 0, unroll=False)
            pltpu.sync_copy(rows_v, out_hbm.at[pl.ds(base + c * chunk, chunk)])
            return carry

        lax.fori_loop(0, nch, body, 0, unroll=False)

    return emb


def kernel(x, embedding_table):
    b, l = x.shape
    v, d = embedding_table.shape
    n = b * l
    nw = _NC * _NS
    chunk = 128
    per_w = n // nw
    nch = per_w // chunk
    scale = float(d) ** 0.5

    idx = x.reshape(nw, nch, chunk).astype(jnp.int32)
    out = _emb_kernel(n, d, nch, chunk, scale)(idx, embedding_table)
    return out.reshape(b, l, d)


# speedup vs baseline: 2.9410x; 1.2144x over previous
import functools

import jax
import jax.numpy as jnp
from jax import lax
from jax.experimental import pallas as pl
from jax.experimental.pallas import tpu as pltpu
from jax.experimental.pallas import tpu_sc as plsc

_NC, _NS = 2, 16
_NBUF = 5


def _emb_kernel(n, d, nch, chunk, scale):
    nw = _NC * _NS
    per_w = n // nw
    ngrp = nch // _NBUF

    mesh = plsc.VectorSubcoreMesh(
        core_axis_name="c", subcore_axis_name="s",
        num_cores=_NC, num_subcores=_NS,
    )

    @functools.partial(
        pl.kernel,
        out_type=jax.ShapeDtypeStruct((n, d), jnp.float32),
        mesh=mesh,
        scratch_types=[
            pltpu.VMEM((nch, chunk), jnp.int32),
            pltpu.VMEM((_NBUF, chunk, d), jnp.float32),
            pltpu.SemaphoreType.DMA((_NBUF,)),
            pltpu.SemaphoreType.DMA((_NBUF,)),
        ],
    )
    def emb(idx_hbm, table_hbm, out_hbm, idx_v, rows_v, gsem, osem):
        wid = lax.axis_index("s") * _NC + lax.axis_index("c")
        base = wid * per_w
        pltpu.sync_copy(idx_hbm.at[wid], idx_v)

        def out_slice(c):
            return out_hbm.at[pl.ds(base + c * chunk, chunk)]

        def group(g, carry):
            c0 = g * _NBUF
            for b in range(_NBUF):
                @pl.when(g > 0)
                def _wait_out(b=b):
                    pltpu.make_async_copy(
                        rows_v.at[b], out_slice(c0 + b), osem.at[b]
                    ).wait()
                pltpu.async_copy(
                    table_hbm.at[idx_v.at[c0 + b]], rows_v.at[b], gsem.at[b]
                )
            for b in range(_NBUF):
                pltpu.make_async_copy(
                    table_hbm.at[idx_v.at[c0 + b]], rows_v.at[b], gsem.at[b]
                ).wait()

                def srow(r, carry2, b=b):
                    for j in range(d // 16):
                        sl = pl.ds(j * 16, 16)
                        rows_v[b, r, sl] = rows_v[b, r, sl] * scale
                    return carry2

                lax.fori_loop(0, chunk, srow, 0, unroll=False)
                pltpu.async_copy(rows_v.at[b], out_slice(c0 + b), osem.at[b])
            return carry

        lax.fori_loop(0, ngrp, group, 0, unroll=False)
        for b in range(_NBUF):
            pltpu.make_async_copy(
                rows_v.at[b], out_slice(b), osem.at[b]
            ).wait()

    return emb


def kernel(x, embedding_table):
    b, l = x.shape
    v, d = embedding_table.shape
    n = b * l
    nw = _NC * _NS
    chunk = 128
    per_w = n // nw
    nch = per_w // chunk
    scale = float(d) ** 0.5

    idx = x.reshape(nw, nch, chunk).astype(jnp.int32)
    out = _emb_kernel(n, d, nch, chunk, scale)(idx, embedding_table)
    return out.reshape(b, l, d)


# speedup vs baseline: 4.9610x; 1.6869x over previous
import functools

import jax
import jax.numpy as jnp
from jax import lax
from jax.experimental import pallas as pl
from jax.experimental.pallas import tpu as pltpu
from jax.experimental.pallas import tpu_sc as plsc

_NC, _NS = 2, 16
_NBUF = 4


def _emb_kernel(bsz, l, d, scale):
    nw = _NC * _NS
    nb = bsz // nw
    ngrp = nb // _NBUF

    mesh = plsc.VectorSubcoreMesh(
        core_axis_name="c", subcore_axis_name="s",
        num_cores=_NC, num_subcores=_NS,
    )

    @functools.partial(
        pl.kernel,
        out_type=jax.ShapeDtypeStruct((bsz, l, d), jnp.float32),
        mesh=mesh,
        scratch_types=[
            pltpu.VMEM((nb, l), jnp.int32),
            pltpu.VMEM((_NBUF, l, d), jnp.float32),
            pltpu.SemaphoreType.DMA((_NBUF,)),
            pltpu.SemaphoreType.DMA((_NBUF,)),
        ],
    )
    def emb(idx_hbm, table_hbm, out_hbm, idx_v, rows_v, gsem, osem):
        wid = lax.axis_index("s") * _NC + lax.axis_index("c")
        base = wid * nb
        pltpu.sync_copy(idx_hbm.at[wid], idx_v)

        def group(g, carry):
            c0 = g * _NBUF
            for b in range(_NBUF):
                @pl.when(g > 0)
                def _wait_out(b=b):
                    pltpu.make_async_copy(
                        rows_v.at[b], out_hbm.at[base + c0 + b], osem.at[b]
                    ).wait()
                pltpu.async_copy(
                    table_hbm.at[idx_v.at[c0 + b]], rows_v.at[b], gsem.at[b]
                )
            for b in range(_NBUF):
                pltpu.make_async_copy(
                    table_hbm.at[idx_v.at[c0 + b]], rows_v.at[b], gsem.at[b]
                ).wait()

                def srow(r, carry2, b=b):
                    for j in range(d // 16):
                        sl = pl.ds(j * 16, 16)
                        rows_v[b, r, sl] = rows_v[b, r, sl] * scale
                    return carry2

                lax.fori_loop(0, l, srow, 0, unroll=False)
                pltpu.async_copy(
                    rows_v.at[b], out_hbm.at[base + c0 + b], osem.at[b]
                )
            return carry

        lax.fori_loop(0, ngrp, group, 0, unroll=False)
        for b in range(_NBUF):
            pltpu.make_async_copy(
                rows_v.at[b], out_hbm.at[base + b], osem.at[b]
            ).wait()

    return emb


def kernel(x, embedding_table):
    bsz, l = x.shape
    v, d = embedding_table.shape
    nw = _NC * _NS
    scale = float(d) ** 0.5

    idx = x.reshape(nw, bsz // nw, l).astype(jnp.int32)
    return _emb_kernel(bsz, l, d, scale)(idx, embedding_table)
